# baseline (speedup 1.0000x reference)
import functools

import jax
import jax.numpy as jnp
from jax import lax
from jax.experimental import pallas as pl
from jax.experimental.pallas import tpu as pltpu
from jax.experimental.pallas import tpu_sc as plsc

_D = 4096
_B = 16384
_NC = 2
_NS = 16
_NW = _NC * _NS
_ROWS_PER_W = _B // _NW
_REP = 31
_NFULL = _ROWS_PER_W // _REP
_TAIL = _ROWS_PER_W - _NFULL * _REP


def _sc_body(idx_hbm, table_hbm, out_hbm, idx_v, buf_v, gsem, ssem):
    wid = lax.axis_index("s") * _NC + lax.axis_index("c")
    base = wid * _ROWS_PER_W
    pltpu.sync_copy(idx_hbm, idx_v)
    pltpu.async_copy(table_hbm.at[idx_v], buf_v, gsem).wait()

    def _fire(j, carry):
        pltpu.async_copy(buf_v, out_hbm.at[pl.ds(base + j * _REP, _REP)], ssem)
        return carry

    lax.fori_loop(0, _NFULL, _fire, 0)
    pltpu.async_copy(
        buf_v.at[pl.ds(0, _TAIL)],
        out_hbm.at[pl.ds(base + _NFULL * _REP, _TAIL)],
        ssem,
    )
    pltpu.make_async_copy(
        out_hbm.at[pl.ds(base, _ROWS_PER_W)],
        out_hbm.at[pl.ds(base, _ROWS_PER_W)],
        ssem,
    ).wait()


def kernel(prefix_emb, variant_idx, batch_size):
    idx = jnp.asarray(variant_idx, jnp.int32) + (
        jnp.asarray(batch_size, jnp.int32) - _B
    )
    idx_arr = jnp.full((_REP,), idx, dtype=jnp.int32)
    table = prefix_emb.reshape(2, 1, _D)
    kfn = functools.partial(
        pl.kernel,
        out_type=jax.ShapeDtypeStruct((_B, 1, _D), jnp.float32),
        mesh=plsc.VectorSubcoreMesh(core_axis_name="c", subcore_axis_name="s"),
        scratch_types=[
            pltpu.VMEM((_REP,), jnp.int32),
            pltpu.VMEM((_REP, 1, _D), jnp.float32),
            pltpu.SemaphoreType.DMA,
            pltpu.SemaphoreType.DMA,
        ],
    )(_sc_body)
    return kfn(idx_arr, table)


# speedup vs baseline: 1.1790x; 1.1790x over previous
import functools

import jax
import jax.numpy as jnp
from jax import lax
from jax.experimental import pallas as pl
from jax.experimental.pallas import tpu as pltpu
from jax.experimental.pallas import tpu_sc as plsc

_D = 4096
_B = 16384
_NC = 2
_NS = 16
_NW = _NC * _NS
_ROWS_PER_W = _B // _NW
_REP = 8
_NCOPY = _ROWS_PER_W // _REP


def _sc_body(idx_hbm, table_hbm, out_hbm, idx_v, buf_v, gsem, ssem):
    wid = lax.axis_index("s") * _NC + lax.axis_index("c")
    base = wid * _ROWS_PER_W
    pltpu.sync_copy(idx_hbm, idx_v)
    pltpu.async_copy(table_hbm.at[idx_v], buf_v, gsem).wait()

    def _fire(j, carry):
        pltpu.async_copy(buf_v, out_hbm.at[pl.ds(base + j * _REP, _REP)], ssem)
        return carry

    lax.fori_loop(0, _NCOPY, _fire, 0)
    pltpu.make_async_copy(
        out_hbm.at[pl.ds(base, _ROWS_PER_W)],
        out_hbm.at[pl.ds(base, _ROWS_PER_W)],
        ssem,
    ).wait()


def kernel(prefix_emb, variant_idx, batch_size):
    idx = jnp.asarray(variant_idx, jnp.int32) + (
        jnp.asarray(batch_size, jnp.int32) - _B
    )
    idx_arr = jnp.full((_REP,), idx, dtype=jnp.int32)
    table = prefix_emb.reshape(2, 1, _D)
    kfn = functools.partial(
        pl.kernel,
        out_type=jax.ShapeDtypeStruct((_B, 1, _D), jnp.float32),
        mesh=plsc.VectorSubcoreMesh(core_axis_name="c", subcore_axis_name="s"),
        scratch_types=[
            pltpu.VMEM((_REP,), jnp.int32),
            pltpu.VMEM((_REP, 1, _D), jnp.float32),
            pltpu.SemaphoreType.DMA,
            pltpu.SemaphoreType.DMA,
        ],
    )(_sc_body)
    return kfn(idx_arr, table)


# speedup vs baseline: 1.2370x; 1.0492x over previous
import functools

import jax
import jax.numpy as jnp
from jax import lax
from jax.experimental import pallas as pl
from jax.experimental.pallas import tpu as pltpu
from jax.experimental.pallas import tpu_sc as plsc

_D = 4096
_B = 16384
_NC = 2
_NS = 16
_NW = _NC * _NS
_ROWS_PER_W = _B // _NW
_REP = 4
_NCOPY = _ROWS_PER_W // _REP


def _sc_body(idx_hbm, table_hbm, out_hbm, idx_v, buf_v, gsem, ssem):
    wid = lax.axis_index("s") * _NC + lax.axis_index("c")
    base = wid * _ROWS_PER_W
    pltpu.sync_copy(idx_hbm, idx_v)
    pltpu.async_copy(table_hbm.at[idx_v], buf_v, gsem).wait()

    def _fire(j, carry):
        pltpu.async_copy(buf_v, out_hbm.at[pl.ds(base + j * _REP, _REP)], ssem)
        return carry

    lax.fori_loop(0, _NCOPY, _fire, 0)
    pltpu.make_async_copy(
        out_hbm.at[pl.ds(base, _ROWS_PER_W)],
        out_hbm.at[pl.ds(base, _ROWS_PER_W)],
        ssem,
    ).wait()


def kernel(prefix_emb, variant_idx, batch_size):
    idx = jnp.asarray(variant_idx, jnp.int32) + (
        jnp.asarray(batch_size, jnp.int32) - _B
    )
    idx_arr = jnp.full((_REP,), idx, dtype=jnp.int32)
    table = prefix_emb.reshape(2, 1, _D)
    kfn = functools.partial(
        pl.kernel,
        out_type=jax.ShapeDtypeStruct((_B, 1, _D), jnp.float32),
        mesh=plsc.VectorSubcoreMesh(core_axis_name="c", subcore_axis_name="s"),
        scratch_types=[
            pltpu.VMEM((_REP,), jnp.int32),
            pltpu.VMEM((_REP, 1, _D), jnp.float32),
            pltpu.SemaphoreType.DMA,
            pltpu.SemaphoreType.DMA,
        ],
    )(_sc_body)
    return kfn(idx_arr, table)


# speedup vs baseline: 1.2791x; 1.0340x over previous
import functools

import jax
import jax.numpy as jnp
from jax import lax
from jax.experimental import pallas as pl
from jax.experimental.pallas import tpu as pltpu
from jax.experimental.pallas import tpu_sc as plsc

_D = 4096
_B = 16384
_NC = 2
_NS = 16
_NW = _NC * _NS
_ROWS_PER_W = _B // _NW
_REP = 2
_NCOPY = _ROWS_PER_W // _REP


def _sc_body(idx_hbm, table_hbm, out_hbm, idx_v, buf_v, gsem, ssem):
    wid = lax.axis_index("s") * _NC + lax.axis_index("c")
    base = wid * _ROWS_PER_W
    pltpu.sync_copy(idx_hbm, idx_v)
    pltpu.async_copy(table_hbm.at[idx_v], buf_v, gsem).wait()

    def _fire(j, carry):
        pltpu.async_copy(buf_v, out_hbm.at[pl.ds(base + j * _REP, _REP)], ssem)
        return carry

    lax.fori_loop(0, _NCOPY, _fire, 0)
    pltpu.make_async_copy(
        out_hbm.at[pl.ds(base, _ROWS_PER_W)],
        out_hbm.at[pl.ds(base, _ROWS_PER_W)],
        ssem,
    ).wait()


def kernel(prefix_emb, variant_idx, batch_size):
    idx = jnp.asarray(variant_idx, jnp.int32) + (
        jnp.asarray(batch_size, jnp.int32) - _B
    )
    idx_arr = jnp.full((_REP,), idx, dtype=jnp.int32)
    table = prefix_emb.reshape(2, 1, _D)
    kfn = functools.partial(
        pl.kernel,
        out_type=jax.ShapeDtypeStruct((_B, 1, _D), jnp.float32),
        mesh=plsc.VectorSubcoreMesh(core_axis_name="c", subcore_axis_name="s"),
        scratch_types=[
            pltpu.VMEM((_REP,), jnp.int32),
            pltpu.VMEM((_REP, 1, _D), jnp.float32),
            pltpu.SemaphoreType.DMA,
            pltpu.SemaphoreType.DMA,
        ],
    )(_sc_body)
    return kfn(idx_arr, table)


# speedup vs baseline: 1.3689x; 1.0702x over previous
import functools

import jax
import jax.numpy as jnp
from jax import lax
from jax.experimental import pallas as pl
from jax.experimental.pallas import tpu as pltpu
from jax.experimental.pallas import tpu_sc as plsc

_D = 4096
_B = 16384
_NC = 2
_NS = 16
_NW = _NC * _NS
_ROWS_PER_W = _B // _NW
_REP = 1
_NCOPY = _ROWS_PER_W // _REP


def _sc_body(idx_hbm, table_hbm, out_hbm, idx_v, buf_v, gsem, ssem):
    wid = lax.axis_index("s") * _NC + lax.axis_index("c")
    base = wid * _ROWS_PER_W
    pltpu.sync_copy(idx_hbm, idx_v)
    pltpu.async_copy(table_hbm.at[idx_v], buf_v, gsem).wait()

    def _fire(j, carry):
        pltpu.async_copy(buf_v, out_hbm.at[pl.ds(base + j * _REP, _REP)], ssem)
        return carry

    lax.fori_loop(0, _NCOPY, _fire, 0)
    pltpu.make_async_copy(
        out_hbm.at[pl.ds(base, _ROWS_PER_W)],
        out_hbm.at[pl.ds(base, _ROWS_PER_W)],
        ssem,
    ).wait()


def kernel(prefix_emb, variant_idx, batch_size):
    idx = jnp.asarray(variant_idx, jnp.int32) + (
        jnp.asarray(batch_size, jnp.int32) - _B
    )
    idx_arr = jnp.full((_REP,), idx, dtype=jnp.int32)
    table = prefix_emb.reshape(2, 1, _D)
    kfn = functools.partial(
        pl.kernel,
        out_type=jax.ShapeDtypeStruct((_B, 1, _D), jnp.float32),
        mesh=plsc.VectorSubcoreMesh(core_axis_name="c", subcore_axis_name="s"),
        scratch_types=[
            pltpu.VMEM((_REP,), jnp.int32),
            pltpu.VMEM((_REP, 1, _D), jnp.float32),
            pltpu.SemaphoreType.DMA,
            pltpu.SemaphoreType.DMA,
        ],
    )(_sc_body)
    return kfn(idx_arr, table)
